# baseline (speedup 1.0000x reference)
import functools

import jax
import jax.numpy as jnp
from jax import lax
from jax.experimental import pallas as pl
from jax.experimental.pallas import tpu as pltpu
from jax.experimental.pallas import tpu_sc as plsc

BATCH = 16384
EMBED_DIM = 64
NUM_WORKERS = 32
BPW = BATCH // NUM_WORKERS
NUM_CORES = 2
CHUNK = 256
NCHUNK = BPW // CHUNK
NPAIR = 500000
PPW = 15624
PCH = 72
NCH = PPW // PCH
PREM = NPAIR - NUM_WORKERS * PPW


def _convert_body(uemb_hbm, iemb_hbm, u2_out, i2_out,
                  in0, in1, out0, out1,
                  si0, si1, so0, so1):
    wid = lax.axis_index("s") * NUM_CORES + lax.axis_index("c")
    p0w = wid * PPW

    ins = (in0, in1)
    outs = (out0, out1)
    sis = (si0, si1)
    sos = (so0, so1)

    for src, dst in ((uemb_hbm, u2_out), (iemb_hbm, i2_out)):

        def in_copy(c, s):
            pltpu.async_copy(
                src.at[pl.ds((p0w + c * PCH) * 2, PCH * 2)], ins[s], sis[s])

        def wait_in(s):
            pltpu.make_async_copy(
                src.at[pl.ds(0, PCH * 2)], ins[s], sis[s]).wait()

        def wait_out(s):
            pltpu.make_async_copy(
                outs[s], dst.at[pl.ds(0, PCH)], sos[s]).wait()

        def repack(s):
            iv, ov = ins[s], outs[s]

            def pair(p, carry):
                for k in range(4):
                    ov[p, pl.ds(k * 16, 16)] = iv[2 * p, pl.ds(k * 16, 16)]
                    ov[p, pl.ds(64 + k * 16, 16)] = (
                        iv[2 * p + 1, pl.ds(k * 16, 16)])
                return carry

            lax.fori_loop(0, PCH, pair, 0)

        def out_copy(c, s):
            pltpu.async_copy(outs[s], dst.at[pl.ds(p0w + c * PCH, PCH)],
                             sos[s])

        in_copy(0, 0)
        in_copy(1, 1)

        def step(t, carry):
            for s in range(2):
                c = 2 * t + s
                wait_in(s)

                @pl.when(t > 0)
                def _():
                    wait_out(s)

                repack(s)
                out_copy(c, s)

                @pl.when(c + 2 < NCH)
                def _():
                    in_copy(c + 2, s)
            return carry

        lax.fori_loop(0, (NCH - 1) // 2, step, 0)

        wait_in(0)
        wait_out(0)
        repack(0)
        out_copy(NCH - 1, 0)

        wait_out(0)
        wait_out(1)

        @pl.when(wid == NUM_WORKERS - 1)
        def _():
            rbase = NUM_WORKERS * PPW
            pltpu.sync_copy(src.at[pl.ds(rbase * 2, PREM * 2)],
                            ins[0].at[pl.ds(0, PREM * 2)])

            def rpair(p, carry):
                for k in range(4):
                    outs[0][p, pl.ds(k * 16, 16)] = (
                        ins[0][2 * p, pl.ds(k * 16, 16)])
                    outs[0][p, pl.ds(64 + k * 16, 16)] = (
                        ins[0][2 * p + 1, pl.ds(k * 16, 16)])
                return carry

            lax.fori_loop(0, PREM, rpair, 0)
            pltpu.sync_copy(outs[0].at[pl.ds(0, PREM)],
                            dst.at[pl.ds(rbase, PREM)])


def _gather_body(uq_hbm, iq_hbm, jq_hbm, uh_hbm, ih_hbm, jh_hbm,
                 uemb2_hbm, iemb2_hbm,
                 pos_out, neg_out,
                 uq_v, iq_v, jq_v, uh_v, ih_v, jh_v,
                 u_rows, i_rows, j_rows,
                 pos_v, neg_v, sem):
    wid = lax.axis_index("s") * NUM_CORES + lax.axis_index("c")
    base = wid * BPW

    pltpu.sync_copy(uq_hbm.at[pl.ds(base, BPW)], uq_v)
    pltpu.sync_copy(iq_hbm.at[pl.ds(base, BPW)], iq_v)
    pltpu.sync_copy(jq_hbm.at[pl.ds(base, BPW)], jq_v)
    pltpu.sync_copy(uh_hbm.at[pl.ds(base, BPW)], uh_v)
    pltpu.sync_copy(ih_hbm.at[pl.ds(base, BPW)], ih_v)
    pltpu.sync_copy(jh_hbm.at[pl.ds(base, BPW)], jh_v)

    lanes = lax.iota(jnp.int32, 16)

    for c in range(NCHUNK):
        co = c * CHUNK
        cu = pltpu.async_copy(uemb2_hbm.at[uq_v.at[pl.ds(co, CHUNK)]],
                              u_rows, sem)
        ci = pltpu.async_copy(iemb2_hbm.at[iq_v.at[pl.ds(co, CHUNK)]],
                              i_rows, sem)
        cj = pltpu.async_copy(iemb2_hbm.at[jq_v.at[pl.ds(co, CHUNK)]],
                              j_rows, sem)
        cu.wait()
        ci.wait()
        cj.wait()

        def group(g, carry, co=co):
            b0 = g * 16
            p_acc = jnp.zeros((16,), jnp.float32)
            n_acc = jnp.zeros((16,), jnp.float32)
            hu_v = uh_v[pl.ds(co + b0, 16)] * 64
            hi_v = ih_v[pl.ds(co + b0, 16)] * 64
            hj_v = jh_v[pl.ds(co + b0, 16)] * 64
            for b in range(16):
                ou = hu_v[b]
                oi = hi_v[b]
                oj = hj_v[b]
                u0 = u_rows[b0 + b, pl.ds(ou, 16)]
                u1 = u_rows[b0 + b, pl.ds(ou + 16, 16)]
                u2 = u_rows[b0 + b, pl.ds(ou + 32, 16)]
                u3 = u_rows[b0 + b, pl.ds(ou + 48, 16)]
                i0 = i_rows[b0 + b, pl.ds(oi, 16)]
                i1 = i_rows[b0 + b, pl.ds(oi + 16, 16)]
                i2 = i_rows[b0 + b, pl.ds(oi + 32, 16)]
                i3 = i_rows[b0 + b, pl.ds(oi + 48, 16)]
                j0 = j_rows[b0 + b, pl.ds(oj, 16)]
                j1 = j_rows[b0 + b, pl.ds(oj + 16, 16)]
                j2 = j_rows[b0 + b, pl.ds(oj + 32, 16)]
                j3 = j_rows[b0 + b, pl.ds(oj + 48, 16)]
                p = (u0 * i0 + u1 * i1) + (u2 * i2 + u3 * i3)
                n = (u0 * j0 + u1 * j1) + (u2 * j2 + u3 * j3)
                sel = lanes == b
                p_acc = jnp.where(sel, jnp.sum(p), p_acc)
                n_acc = jnp.where(sel, jnp.sum(n), n_acc)
            pos_v[pl.ds(co + b0, 16)] = p_acc
            neg_v[pl.ds(co + b0, 16)] = n_acc
            return carry

        lax.fori_loop(0, CHUNK // 16, group, 0)

    pltpu.sync_copy(pos_v, pos_out.at[pl.ds(base, BPW)])
    pltpu.sync_copy(neg_v, neg_out.at[pl.ds(base, BPW)])


def _repack_tc_body(a_ref, b_ref, out_ref):
    out_ref[:, pl.ds(0, EMBED_DIM)] = a_ref[...]
    out_ref[:, pl.ds(EMBED_DIM, EMBED_DIM)] = b_ref[...]


def _tc_repack(x):
    return pl.pallas_call(
        _repack_tc_body,
        grid=(500,),
        in_specs=[
            pl.BlockSpec((1000, EMBED_DIM), lambda i: (i, 0)),
            pl.BlockSpec((1000, EMBED_DIM), lambda i: (i + 500, 0)),
        ],
        out_specs=pl.BlockSpec((1000, 2 * EMBED_DIM), lambda i: (i, 0)),
        out_shape=jax.ShapeDtypeStruct((NPAIR, 2 * EMBED_DIM), jnp.float32),
    )(x, x)


@jax.jit
def kernel(user, pos_item, neg_item, user_emb, item_emb):
    mesh = plsc.VectorSubcoreMesh(core_axis_name="c", subcore_axis_name="s")
    params = pltpu.CompilerParams(needs_layout_passes=False)

    u2 = _tc_repack(user_emb)
    i2 = _tc_repack(item_emb)

    half = jnp.int32(NPAIR)
    uh = (user >= half).astype(jnp.int32)
    ih = (pos_item >= half).astype(jnp.int32)
    jh = (neg_item >= half).astype(jnp.int32)
    uq = user - uh * half
    iq = pos_item - ih * half
    jq = neg_item - jh * half

    gather = pl.kernel(
        _gather_body,
        mesh=mesh,
        compiler_params=params,
        out_type=(
            jax.ShapeDtypeStruct((BATCH,), jnp.float32),
            jax.ShapeDtypeStruct((BATCH,), jnp.float32),
        ),
        scratch_types=[
            pltpu.VMEM((BPW,), jnp.int32),
            pltpu.VMEM((BPW,), jnp.int32),
            pltpu.VMEM((BPW,), jnp.int32),
            pltpu.VMEM((BPW,), jnp.int32),
            pltpu.VMEM((BPW,), jnp.int32),
            pltpu.VMEM((BPW,), jnp.int32),
            pltpu.VMEM((CHUNK, 128), jnp.float32),
            pltpu.VMEM((CHUNK, 128), jnp.float32),
            pltpu.VMEM((CHUNK, 128), jnp.float32),
            pltpu.VMEM((BPW,), jnp.float32),
            pltpu.VMEM((BPW,), jnp.float32),
            pltpu.SemaphoreType.DMA,
        ],
    )
    return gather(uq, iq, jq, uh, ih, jh, u2, i2)


# speedup vs baseline: 2.2514x; 2.2514x over previous
import functools

import jax
import jax.numpy as jnp
from jax import lax
from jax.experimental import pallas as pl
from jax.experimental.pallas import tpu as pltpu
from jax.experimental.pallas import tpu_sc as plsc

BATCH = 16384
EMBED_DIM = 64
NUM_WORKERS = 32
BPW = BATCH // NUM_WORKERS
NUM_CORES = 2
CHUNK = 256
NCHUNK = BPW // CHUNK


def _bprmf_body(user_hbm, pos_hbm, neg_hbm, uemb_hbm, iemb_hbm,
                pos_out, neg_out,
                uq_v, iq_v, jq_v,
                u_rows, i_rows, j_rows,
                pos_v, neg_v, semu, semi, semj):
    wid = lax.axis_index("s") * NUM_CORES + lax.axis_index("c")
    base = wid * BPW

    pltpu.sync_copy(user_hbm.at[pl.ds(base, BPW)], uq_v)
    pltpu.sync_copy(pos_hbm.at[pl.ds(base, BPW)], iq_v)
    pltpu.sync_copy(neg_hbm.at[pl.ds(base, BPW)], jq_v)

    lanes = lax.iota(jnp.int32, 16)

    for c in range(NCHUNK):
        co = c * CHUNK

        def fire(g, carry, co=co):
            b0 = g * 16
            ru = uq_v[pl.ds(co + b0, 16)]
            ri = iq_v[pl.ds(co + b0, 16)]
            rj = jq_v[pl.ds(co + b0, 16)]
            for b in range(16):
                pltpu.async_copy(uemb_hbm.at[pl.ds(ru[b], 1)],
                                 u_rows.at[pl.ds(b0 + b, 1)], semu)
                pltpu.async_copy(iemb_hbm.at[pl.ds(ri[b], 1)],
                                 i_rows.at[pl.ds(b0 + b, 1)], semi)
                pltpu.async_copy(iemb_hbm.at[pl.ds(rj[b], 1)],
                                 j_rows.at[pl.ds(b0 + b, 1)], semj)
            return carry

        lax.fori_loop(0, CHUNK // 16, fire, 0)

        pltpu.make_async_copy(uemb_hbm.at[pl.ds(0, CHUNK)], u_rows,
                              semu).wait()
        pltpu.make_async_copy(uemb_hbm.at[pl.ds(0, CHUNK)], i_rows,
                              semi).wait()
        pltpu.make_async_copy(uemb_hbm.at[pl.ds(0, CHUNK)], j_rows,
                              semj).wait()

        def group(g, carry, co=co):
            b0 = g * 16
            p_acc = jnp.zeros((16,), jnp.float32)
            n_acc = jnp.zeros((16,), jnp.float32)
            for b in range(16):
                u0 = u_rows[b0 + b, pl.ds(0, 16)]
                u1 = u_rows[b0 + b, pl.ds(16, 16)]
                u2 = u_rows[b0 + b, pl.ds(32, 16)]
                u3 = u_rows[b0 + b, pl.ds(48, 16)]
                i0 = i_rows[b0 + b, pl.ds(0, 16)]
                i1 = i_rows[b0 + b, pl.ds(16, 16)]
                i2 = i_rows[b0 + b, pl.ds(32, 16)]
                i3 = i_rows[b0 + b, pl.ds(48, 16)]
                j0 = j_rows[b0 + b, pl.ds(0, 16)]
                j1 = j_rows[b0 + b, pl.ds(16, 16)]
                j2 = j_rows[b0 + b, pl.ds(32, 16)]
                j3 = j_rows[b0 + b, pl.ds(48, 16)]
                p = (u0 * i0 + u1 * i1) + (u2 * i2 + u3 * i3)
                n = (u0 * j0 + u1 * j1) + (u2 * j2 + u3 * j3)
                sel = lanes == b
                p_acc = jnp.where(sel, jnp.sum(p), p_acc)
                n_acc = jnp.where(sel, jnp.sum(n), n_acc)
            pos_v[pl.ds(co + b0, 16)] = p_acc
            neg_v[pl.ds(co + b0, 16)] = n_acc
            return carry

        lax.fori_loop(0, CHUNK // 16, group, 0)

    pltpu.sync_copy(pos_v, pos_out.at[pl.ds(base, BPW)])
    pltpu.sync_copy(neg_v, neg_out.at[pl.ds(base, BPW)])


@jax.jit
def kernel(user, pos_item, neg_item, user_emb, item_emb):
    mesh = plsc.VectorSubcoreMesh(core_axis_name="c", subcore_axis_name="s")
    f = pl.kernel(
        _bprmf_body,
        mesh=mesh,
        compiler_params=pltpu.CompilerParams(needs_layout_passes=False),
        out_type=(
            jax.ShapeDtypeStruct((BATCH,), jnp.float32),
            jax.ShapeDtypeStruct((BATCH,), jnp.float32),
        ),
        scratch_types=[
            pltpu.VMEM((BPW,), jnp.int32),
            pltpu.VMEM((BPW,), jnp.int32),
            pltpu.VMEM((BPW,), jnp.int32),
            pltpu.VMEM((CHUNK, EMBED_DIM), jnp.float32),
            pltpu.VMEM((CHUNK, EMBED_DIM), jnp.float32),
            pltpu.VMEM((CHUNK, EMBED_DIM), jnp.float32),
            pltpu.VMEM((BPW,), jnp.float32),
            pltpu.VMEM((BPW,), jnp.float32),
            pltpu.SemaphoreType.DMA,
            pltpu.SemaphoreType.DMA,
            pltpu.SemaphoreType.DMA,
        ],
    )
    return f(user, pos_item, neg_item, user_emb, item_emb)
